# baseline (speedup 1.0000x reference)
import functools

import jax
import jax.numpy as jnp
from jax import lax
from jax.experimental import pallas as pl
from jax.experimental.pallas import tpu as pltpu
from jax.experimental.pallas import tpu_sc as plsc

_REG = 1e-12

_NC = 2
_NS = 16
_NW = _NC * _NS
_L = 16

_BATCH = 16384
_D = 64
_BPW = _BATCH // _NW
_CH = 32
_NCHUNK = _BPW // _CH
_GPC = _CH // _L


def _sc_body(uids_hbm, iids_hbm, utab_hbm, itab_hbm,
             out_hbm, ssqu_hbm, ssqi_hbm,
             uid_v, iid_v, ublk_v, iblk_v,
             dots_v, ssq_v, sem):
    wid = lax.axis_index("s") * _NC + lax.axis_index("c")

    pltpu.sync_copy(uids_hbm.at[wid], uid_v)
    pltpu.sync_copy(iids_hbm.at[wid], iid_v)

    lanes = lax.broadcasted_iota(jnp.int32, (_L,), 0)
    zeros = jnp.zeros((_L,), jnp.float32)

    def chunk_body(c, carry):
        su, si = carry
        cbase = c * _CH

        for h in range(_GPC):
            uv = lax.shift_right_logical(uid_v[pl.ds(cbase + h * _L, _L)], 3)
            iv = lax.shift_right_logical(iid_v[pl.ds(cbase + h * _L, _L)], 3)
            for l in range(_L):
                m = h * _L + l
                pltpu.async_copy(utab_hbm.at[uv[l]], ublk_v.at[m], sem)
                pltpu.async_copy(itab_hbm.at[iv[l]], iblk_v.at[m], sem)
        for _ in range(2 * _CH):
            pltpu.make_async_copy(utab_hbm.at[0], ublk_v.at[0], sem).wait()

        for g in range(_GPC):
            uv = uid_v[pl.ds(cbase + g * _L, _L)]
            iv = iid_v[pl.ds(cbase + g * _L, _L)]
            ru = jnp.bitwise_and(uv, 7)
            ri = jnp.bitwise_and(iv, 7)
            m = g * _L + lanes
            acc = zeros
            for d in range(_D):
                cols = jnp.full((_L,), d, jnp.int32)
                u = plsc.load_gather(ublk_v, [m, ru, cols])
                v = plsc.load_gather(iblk_v, [m, ri, cols])
                acc = acc + u * v
                su = su + u * u
                si = si + v * v
            dots_v[pl.ds(cbase + g * _L, _L)] = acc
        return (su, si)

    su, si = lax.fori_loop(0, _NCHUNK, chunk_body, (zeros, zeros))
    ssq_v[0, :] = su
    ssq_v[1, :] = si

    pltpu.sync_copy(dots_v, out_hbm.at[pl.ds(wid * _BPW, _BPW)])
    pltpu.sync_copy(ssq_v.at[0], ssqu_hbm.at[wid])
    pltpu.sync_copy(ssq_v.at[1], ssqi_hbm.at[wid])


@functools.partial(
    pl.kernel,
    out_type=(
        jax.ShapeDtypeStruct((_BATCH,), jnp.float32),
        jax.ShapeDtypeStruct((_NW, _L), jnp.float32),
        jax.ShapeDtypeStruct((_NW, _L), jnp.float32),
    ),
    mesh=plsc.VectorSubcoreMesh(core_axis_name="c", subcore_axis_name="s"),
    compiler_params=pltpu.CompilerParams(needs_layout_passes=False),
    scratch_types=(
        pltpu.VMEM((_BPW,), jnp.int32),
        pltpu.VMEM((_BPW,), jnp.int32),
        pltpu.VMEM((_CH, 8, _D), jnp.float32),
        pltpu.VMEM((_CH, 8, _D), jnp.float32),
        pltpu.VMEM((_BPW,), jnp.float32),
        pltpu.VMEM((2, _L), jnp.float32),
        pltpu.SemaphoreType.DMA,
    ),
)
def _funk_svd_sc(uids_hbm, iids_hbm, utab_hbm, itab_hbm,
                 out_hbm, ssqu_hbm, ssqi_hbm, *scratch):
    _sc_body(uids_hbm, iids_hbm, utab_hbm, itab_hbm,
             out_hbm, ssqu_hbm, ssqi_hbm, *scratch)


def kernel(user_ids, item_ids, user_table, item_table):
    utab = user_table.reshape(125000, 8, _D)
    itab = item_table.reshape(125000, 8, _D)
    uids = user_ids.reshape(_NW, _BPW)
    iids = item_ids.reshape(_NW, _BPW)
    dots, ssqu, ssqi = _funk_svd_sc(uids, iids, utab, itab)
    reg = _REG * (jnp.sqrt(jnp.sum(ssqu)) + jnp.sqrt(jnp.sum(ssqi)))
    return dots + reg


# speedup vs baseline: 1.0025x; 1.0025x over previous
import functools

import jax
import jax.numpy as jnp
from jax import lax
from jax.experimental import pallas as pl
from jax.experimental.pallas import tpu as pltpu
from jax.experimental.pallas import tpu_sc as plsc

_REG = 1e-12

_NC = 2
_NS = 16
_NW = _NC * _NS
_L = 16

_BATCH = 16384
_D = 64
_BPW = _BATCH // _NW
_CH = 16
_NCHUNK = _BPW // _CH
_GPC = _CH // _L


def _sc_body(uids_hbm, iids_hbm, utab_hbm, itab_hbm,
             out_hbm, ssqu_hbm, ssqi_hbm,
             uid_v, iid_v, ublk_a, iblk_a, ublk_b, iblk_b,
             dots_v, ssq_v, sem_a, sem_b):
    wid = lax.axis_index("s") * _NC + lax.axis_index("c")

    pltpu.sync_copy(uids_hbm.at[wid], uid_v)
    pltpu.sync_copy(iids_hbm.at[wid], iid_v)

    lanes = lax.broadcasted_iota(jnp.int32, (_L,), 0)
    zeros = jnp.zeros((_L,), jnp.float32)

    def fire(cbase, ublk, iblk, sem):
        for h in range(_GPC):
            uv = lax.shift_right_logical(uid_v[pl.ds(cbase + h * _L, _L)], 3)
            iv = lax.shift_right_logical(iid_v[pl.ds(cbase + h * _L, _L)], 3)
            for l in range(_L):
                m = h * _L + l
                pltpu.async_copy(utab_hbm.at[uv[l]], ublk.at[m], sem)
                pltpu.async_copy(itab_hbm.at[iv[l]], iblk.at[m], sem)

    def drain(ublk, iblk, sem):
        pltpu.make_async_copy(utab_hbm.at[pl.ds(0, _CH)], ublk, sem).wait()
        pltpu.make_async_copy(itab_hbm.at[pl.ds(0, _CH)], iblk, sem).wait()

    def extract(cbase, ublk, iblk, su, si):
        for g in range(_GPC):
            uv = uid_v[pl.ds(cbase + g * _L, _L)]
            iv = iid_v[pl.ds(cbase + g * _L, _L)]
            ru = jnp.bitwise_and(uv, 7)
            ri = jnp.bitwise_and(iv, 7)
            m = g * _L + lanes
            acc = zeros
            for d in range(_D):
                cols = jnp.full((_L,), d, jnp.int32)
                u = plsc.load_gather(ublk, [m, ru, cols])
                v = plsc.load_gather(iblk, [m, ri, cols])
                acc = acc + u * v
                su = su + u * u
                si = si + v * v
            dots_v[pl.ds(cbase + g * _L, _L)] = acc
        return su, si

    def pair_body(p, carry):
        su, si = carry
        ca = (2 * p) * _CH
        cb = (2 * p + 1) * _CH
        fire(ca, ublk_a, iblk_a, sem_a)
        fire(cb, ublk_b, iblk_b, sem_b)
        drain(ublk_a, iblk_a, sem_a)
        su, si = extract(ca, ublk_a, iblk_a, su, si)
        drain(ublk_b, iblk_b, sem_b)
        su, si = extract(cb, ublk_b, iblk_b, su, si)
        return (su, si)

    su, si = lax.fori_loop(0, _NCHUNK // 2, pair_body, (zeros, zeros))
    ssq_v[0, :] = su
    ssq_v[1, :] = si

    pltpu.sync_copy(dots_v, out_hbm.at[pl.ds(wid * _BPW, _BPW)])
    pltpu.sync_copy(ssq_v.at[0], ssqu_hbm.at[wid])
    pltpu.sync_copy(ssq_v.at[1], ssqi_hbm.at[wid])


@functools.partial(
    pl.kernel,
    out_type=(
        jax.ShapeDtypeStruct((_BATCH,), jnp.float32),
        jax.ShapeDtypeStruct((_NW, _L), jnp.float32),
        jax.ShapeDtypeStruct((_NW, _L), jnp.float32),
    ),
    mesh=plsc.VectorSubcoreMesh(core_axis_name="c", subcore_axis_name="s"),
    compiler_params=pltpu.CompilerParams(needs_layout_passes=False),
    scratch_types=(
        pltpu.VMEM((_BPW,), jnp.int32),
        pltpu.VMEM((_BPW,), jnp.int32),
        pltpu.VMEM((_CH, 8, _D), jnp.float32),
        pltpu.VMEM((_CH, 8, _D), jnp.float32),
        pltpu.VMEM((_CH, 8, _D), jnp.float32),
        pltpu.VMEM((_CH, 8, _D), jnp.float32),
        pltpu.VMEM((_BPW,), jnp.float32),
        pltpu.VMEM((2, _L), jnp.float32),
        pltpu.SemaphoreType.DMA,
        pltpu.SemaphoreType.DMA,
    ),
)
def _funk_svd_sc(uids_hbm, iids_hbm, utab_hbm, itab_hbm,
                 out_hbm, ssqu_hbm, ssqi_hbm, *scratch):
    _sc_body(uids_hbm, iids_hbm, utab_hbm, itab_hbm,
             out_hbm, ssqu_hbm, ssqi_hbm, *scratch)


def kernel(user_ids, item_ids, user_table, item_table):
    utab = user_table.reshape(125000, 8, _D)
    itab = item_table.reshape(125000, 8, _D)
    uids = user_ids.reshape(_NW, _BPW)
    iids = item_ids.reshape(_NW, _BPW)
    dots, ssqu, ssqi = _funk_svd_sc(uids, iids, utab, itab)
    reg = _REG * (jnp.sqrt(jnp.sum(ssqu)) + jnp.sqrt(jnp.sum(ssqi)))
    return dots + reg
